# Initial kernel scaffold; baseline (speedup 1.0000x reference)
import jax
import jax.numpy as jnp
from jax.experimental import pallas as pl


def kernel(node_m1_input, node_m2_input, edge_input, angle_input, node_router_weights, node_router_indices, edge_router_weights, edge_router_indices, angle_router_weights, angle_router_indices, n2e_index, n2a_index, node_self_W, node_self_b, node_self_Wsh, node_self_bsh, node_sym_W, node_sym_b, node_sym_Wsh, node_sym_bsh, edge_W, edge_b, edge_Wsh, edge_bsh, angle_W, angle_b, angle_Wsh, angle_bsh):
    raise NotImplementedError("write your pallas kernel here")


# speedup vs baseline: 1.4811x; 1.4811x over previous
import functools

import jax
import jax.numpy as jnp
from jax.experimental import pallas as pl


def _moe_block_kernel(x_ref, w_ref, tw_ref, tidx_ref, b_ref, out_ref):
    e = pl.program_id(1)
    n_exp = pl.num_programs(1)

    @pl.when(e == 0)
    def _init():
        out_ref[...] = jnp.zeros_like(out_ref)

    y = jnp.dot(x_ref[...], w_ref[0], preferred_element_type=jnp.float32)
    y = y + b_ref[0]
    y = y * jax.nn.sigmoid(y)

    tw = tw_ref[...]
    tidx = tidx_ref[...]
    wgt = jnp.sum(tw * (tidx == e).astype(jnp.float32), axis=1)
    wgt = jnp.where(e == n_exp - 1, 1.0, wgt)
    out_ref[...] += wgt[:, None] * y


def _moe_stream(x, topk_w, topk_idx, W, b, Wsh, bsh, blk_rows):
    n, din = x.shape
    dout = W.shape[-1]
    n_exp = W.shape[0] + 1
    w_full = jnp.concatenate([W, Wsh[None]], axis=0)
    b_full = jnp.concatenate([b, bsh[None]], axis=0)[:, None, :]
    grid = (n // blk_rows, n_exp)
    return pl.pallas_call(
        _moe_block_kernel,
        grid=grid,
        in_specs=[
            pl.BlockSpec((blk_rows, din), lambda i, e: (i, 0)),
            pl.BlockSpec((1, din, dout), lambda i, e: (e, 0, 0)),
            pl.BlockSpec((blk_rows, topk_w.shape[1]), lambda i, e: (i, 0)),
            pl.BlockSpec((blk_rows, topk_idx.shape[1]), lambda i, e: (i, 0)),
            pl.BlockSpec((1, 1, dout), lambda i, e: (e, 0, 0)),
        ],
        out_specs=pl.BlockSpec((blk_rows, dout), lambda i, e: (i, 0)),
        out_shape=jax.ShapeDtypeStruct((n, dout), jnp.float32),
    )(x, w_full, topk_w, topk_idx, b_full)


@jax.jit
def kernel(node_m1_input, node_m2_input, edge_input, angle_input,
           node_router_weights, node_router_indices,
           edge_router_weights, edge_router_indices,
           angle_router_weights, angle_router_indices,
           n2e_index, n2a_index,
           node_self_W, node_self_b, node_self_Wsh, node_self_bsh,
           node_sym_W, node_sym_b, node_sym_Wsh, node_sym_bsh,
           edge_W, edge_b, edge_Wsh, edge_bsh,
           angle_W, angle_b, angle_Wsh, angle_bsh):
    edge_idx = edge_router_indices[n2e_index]
    angle_idx = angle_router_indices[n2a_index]
    edge_w = edge_router_weights[n2e_index]
    angle_w = angle_router_weights[n2a_index]

    node_self_out = _moe_stream(node_m1_input, node_router_weights,
                                node_router_indices, node_self_W, node_self_b,
                                node_self_Wsh, node_self_bsh, 256)
    node_sym_out = _moe_stream(node_m2_input, node_router_weights,
                               node_router_indices, node_sym_W, node_sym_b,
                               node_sym_Wsh, node_sym_bsh, 256)
    edge_out = _moe_stream(edge_input, edge_w, edge_idx, edge_W, edge_b,
                           edge_Wsh, edge_bsh, 256)
    angle_out = _moe_stream(angle_input, angle_w, angle_idx, angle_W, angle_b,
                            angle_Wsh, angle_bsh, 256)
    return node_self_out, node_sym_out, edge_out, angle_out
